# baseline (speedup 1.0000x reference)
import functools

import jax
import jax.numpy as jnp
from jax.experimental import pallas as pl
from jax.experimental.pallas import tpu as pltpu

M = 4096
K = 128
OUT = 64
BM = 256
BN = 128
_F32 = jnp.float32
_PREC = jax.lax.Precision.DEFAULT


def _cross_kernel(bd_ref, bs_ref, xd_ref, xs_ref, w_ref, out_ref):
    bd = bd_ref[...]
    bs = bs_ref[...]
    xd = xd_ref[...]
    gmin = bd[0, 0]
    gmax = bd[BM - 1, 0]
    start = jnp.sum((bs < gmin).astype(_F32)).astype(jnp.int32)
    end = jnp.sum((bs <= gmax).astype(_F32)).astype(jnp.int32)
    bn = jnp.int32(BN)
    c0 = jax.lax.div(start, bn)
    c1 = jax.lax.div(end + bn - jnp.int32(1), bn)
    dnorm = jnp.sqrt(jnp.sum(xd * xd, axis=1, keepdims=True))
    cden = _F32(1e-6) * dnorm

    def body(c, carry):
        acc, s = carry
        off = c * jnp.int32(BN)
        xs = xs_ref[pl.ds(off, BN), :]
        bsb = bs_ref[:, pl.ds(off, BN)]
        ss = jnp.sum(xs * xs, axis=1, keepdims=True)
        sst = jnp.transpose(ss)
        rs = jnp.where(sst < 1e-12, _F32(1e6), jax.lax.rsqrt(sst))
        p = jax.lax.dot_general(xd, xs, (((1,), (1,)), ((), ())),
                                precision=_PREC)
        q = jnp.maximum(p, 0.0) * rs
        mask = bd == bsb
        cm = jnp.where(mask, q, 0.0)
        w_inc = jnp.where(mask, q + cden, 0.0)
        acc = acc + jax.lax.dot_general(cm, xs, (((1,), (0,)), ((), ())),
                                        precision=_PREC)
        s = s + jnp.sum(w_inc, axis=1, keepdims=True)
        return acc, s

    acc0 = jnp.zeros((BM, K), _F32)
    s0 = jnp.zeros((BM, 1), _F32)
    acc, s = jax.lax.fori_loop(c0, c1, body, (acc0, s0))

    rinv = jnp.where(s > 0, 1.0 / s, 0.0)
    gx = acc * rinv
    w2 = w_ref[...]
    w2 = w2 * w2
    dot = lambda a: jax.lax.dot_general(a, w2, (((1,), (1,)), ((), ())),
                                        precision=_PREC)
    num = dot(xd * gx)
    td = jnp.sqrt(dot(xd * xd) + 1e-6)
    gd = jnp.sqrt(dot(gx * gx) + 1e-6)
    out_ref[...] = num / jnp.maximum(td * gd, 1e-6)


def _z():
    return jnp.int32(0)


@functools.partial(jax.jit, static_argnames=("interpret",))
def _run(x_left, bl, x_right, br, weight, interpret=False):
    grid = (M // BM,)
    call = functools.partial(
        pl.pallas_call,
        grid=grid,
        out_shape=jax.ShapeDtypeStruct((M, OUT), _F32),
        in_specs=[
            pl.BlockSpec((BM, 1), lambda i: (i, _z())),
            pl.BlockSpec((1, M), lambda i: (_z(), _z())),
            pl.BlockSpec((BM, K), lambda i: (i, _z())),
            pl.BlockSpec((M, K), lambda i: (_z(), _z())),
            pl.BlockSpec((OUT, K), lambda i: (_z(), _z())),
        ],
        out_specs=pl.BlockSpec((BM, OUT), lambda i: (i, _z())),
        compiler_params=pltpu.CompilerParams(dimension_semantics=("parallel",)),
        interpret=interpret,
    )
    out1 = call(_cross_kernel)(bl[:, None], br[None, :], x_left, x_right, weight)
    out2 = call(_cross_kernel)(br[:, None], bl[None, :], x_right, x_left, weight)
    return out1, out2


def kernel(x_left, batch_left, x_right, batch_right, weight):
    bl = batch_left.astype(jnp.float32)
    br = batch_right.astype(jnp.float32)
    return _run(x_left, bl, x_right, br, weight)


# speedup vs baseline: 1.1973x; 1.1973x over previous
import functools

import jax
import jax.numpy as jnp
from jax.experimental import pallas as pl

M = 4096
K = 128
OUT = 64
BM = 256
BN = 128
_F32 = jnp.float32
_PREC = jax.lax.Precision.DEFAULT
_NBLK = M // BN


def _window(bd, bs_row):
    gmin = bd[0, 0]
    gmax = bd[BM - 1, 0]
    start = jnp.sum((bs_row < gmin).astype(_F32)).astype(jnp.int32)
    end = jnp.sum((bs_row <= gmax).astype(_F32)).astype(jnp.int32)
    bn = jnp.int32(BN)
    c0 = jax.lax.div(start, bn)
    c1 = jax.lax.div(end + bn - jnp.int32(1), bn)
    return c0, c1 - c0


def _step(c, xd, bd, xs_ref, bsr_ref, c0, n, cden, acc, s):
    cc = c0 + jnp.clip(c, jnp.int32(0), jnp.maximum(n - jnp.int32(1), jnp.int32(0)))
    cc = jnp.minimum(cc, jnp.int32(_NBLK - 1))
    off = cc * jnp.int32(BN)
    xs = xs_ref[pl.ds(off, BN), :]
    bsb = bsr_ref[:, pl.ds(off, BN)]
    ss = jnp.sum(xs * xs, axis=1, keepdims=True)
    rs = jnp.where(ss.T < 1e-12, _F32(1e6), jax.lax.rsqrt(ss.T))
    p = jax.lax.dot_general(xd, xs, (((1,), (1,)), ((), ())),
                            precision=_PREC)
    q = jnp.maximum(p, 0.0) * rs
    mask = jnp.logical_and(bd == bsb, c < n)
    cm = jnp.where(mask, q, 0.0)
    w_inc = jnp.where(mask, q + cden, 0.0)
    acc = acc + jax.lax.dot_general(cm, xs, (((1,), (0,)), ((), ())),
                                    precision=_PREC)
    s = s + jnp.sum(w_inc, axis=1, keepdims=True)
    return acc, s


def _combine(xd, acc, s, w2):
    rinv = jnp.where(s > 0, 1.0 / s, 0.0)
    gx = acc * rinv
    dot = lambda a: jax.lax.dot_general(a, w2, (((1,), (1,)), ((), ())),
                                        precision=_PREC)
    num = dot(xd * gx)
    td = jnp.sqrt(dot(xd * xd) + 1e-6)
    gd = jnp.sqrt(dot(gx * gx) + 1e-6)
    return num / jnp.maximum(td * gd, 1e-6)


def _cross_kernel(blc_ref, brc_ref, blr_ref, brr_ref, xl_ref, xr_ref, w_ref,
                  o1_ref, o2_ref):
    offd = pl.program_id(0) * jnp.int32(BM)
    bdl = blc_ref[pl.ds(offd, BM), :]
    bdr = brc_ref[pl.ds(offd, BM), :]
    xdl = xl_ref[pl.ds(offd, BM), :]
    xdr = xr_ref[pl.ds(offd, BM), :]

    c0a, na = _window(bdl, brr_ref[...])
    c0b, nb = _window(bdr, blr_ref[...])
    dnl = jnp.sqrt(jnp.sum(xdl * xdl, axis=1, keepdims=True))
    dnr = jnp.sqrt(jnp.sum(xdr * xdr, axis=1, keepdims=True))
    cdl = _F32(1e-6) * dnl
    cdr = _F32(1e-6) * dnr

    def body(c, carry):
        acc_a, s_a, acc_b, s_b = carry
        acc_a, s_a = _step(c, xdl, bdl, xr_ref, brr_ref, c0a, na, cdl, acc_a, s_a)
        acc_b, s_b = _step(c, xdr, bdr, xl_ref, blr_ref, c0b, nb, cdr, acc_b, s_b)
        return acc_a, s_a, acc_b, s_b

    z_acc = jnp.zeros((BM, K), _F32)
    z_s = jnp.zeros((BM, 1), _F32)
    n = jnp.maximum(na, nb)
    acc_a, s_a, acc_b, s_b = jax.lax.fori_loop(
        jnp.int32(0), n, body, (z_acc, z_s, z_acc, z_s))

    w2 = w_ref[...]
    w2 = w2 * w2
    o1_ref[...] = _combine(xdl, acc_a, s_a, w2)
    o2_ref[...] = _combine(xdr, acc_b, s_b, w2)


def _z():
    return jnp.int32(0)


@functools.partial(jax.jit, static_argnames=("interpret",))
def _run(x_left, bl, x_right, br, weight, interpret=False):
    grid = (M // BM,)
    full = lambda shape: pl.BlockSpec(shape, lambda i: (_z(), _z()))
    out_spec = pl.BlockSpec((BM, OUT), lambda i: (i, _z()))
    out1, out2 = pl.pallas_call(
        _cross_kernel,
        grid=grid,
        out_shape=[jax.ShapeDtypeStruct((M, OUT), _F32),
                   jax.ShapeDtypeStruct((M, OUT), _F32)],
        in_specs=[
            full((M, 1)),
            full((M, 1)),
            full((1, M)),
            full((1, M)),
            full((M, K)),
            full((M, K)),
            full((OUT, K)),
        ],
        out_specs=[out_spec, out_spec],
        interpret=interpret,
    )(bl[:, None], br[:, None], bl[None, :], br[None, :],
      x_left, x_right, weight)
    return out1, out2


def kernel(x_left, batch_left, x_right, batch_right, weight):
    bl = batch_left.astype(jnp.float32)
    br = batch_right.astype(jnp.float32)
    return _run(x_left, bl, x_right, br, weight)
